# baseline (speedup 1.0000x reference)
import functools

import jax
import jax.numpy as jnp
from jax import lax
from jax.experimental import pallas as pl
from jax.experimental.pallas import tpu as pltpu
from jax.experimental.pallas import tpu_sc as plsc

BATCH = 16384
LATENT = 32
HIDDEN = 128
NUM_ROWS = 1000000
VP = 131072
SBITS = 17
NS_GROUP = 8
RBLK = 4096
NBLK = VP // RBLK
MAXBLK = (NUM_ROWS - 1) // RBLK


def _repack_body(*refs):
    srefs, out = refs[:NS_GROUP], refs[NS_GROUP]
    k = NS_GROUP * LATENT
    frow = lax.broadcasted_iota(jnp.int32, (k, 128), 0)
    jcol = lax.broadcasted_iota(jnp.int32, (k, 128), 1)
    s = frow // LATENT
    f = frow % LATENT
    e_lo = ((f < 16) & (jcol == s * 16 + f)).astype(jnp.bfloat16)
    e_hi = ((f >= 16) & (jcol == s * 16 + f - 16)).astype(jnp.bfloat16)
    dnums = (((0,), (0,)), ((), ()))

    x = jnp.concatenate([r[...] for r in srefs],
                        axis=0).astype(jnp.bfloat16)
    lo = lax.dot_general(x, e_lo, dnums,
                         preferred_element_type=jnp.float32)
    hi = lax.dot_general(x, e_hi, dnums,
                         preferred_element_type=jnp.float32)
    lo_b = lax.bitcast_convert_type(lo, jnp.uint32)
    hi_b = lax.bitcast_convert_type(hi, jnp.uint32)
    pair = (hi_b & jnp.uint32(0xFFFF0000)) | (lo_b >> 16)
    out[...] = lax.bitcast_convert_type(pair, jnp.float32)


_REPACK = pl.pallas_call(
    _repack_body,
    grid=(NBLK,),
    in_specs=[pl.BlockSpec(
        (LATENT, RBLK),
        lambda i, s=s: (0, jnp.minimum(s * NBLK + i, MAXBLK)))
        for s in range(NS_GROUP)],
    out_specs=pl.BlockSpec((RBLK, 128), lambda i: (i, 0)),
    out_shape=jax.ShapeDtypeStruct((VP, 128), jnp.float32),
)


def _build_gather():
    info = plsc.get_sparse_core_info()
    nc, ns = info.num_cores, info.num_subcores
    nw = nc * ns
    b_per_w = BATCH // nw
    n_chunks = b_per_w // 128
    mesh = plsc.VectorSubcoreMesh(core_axis_name="c", subcore_axis_name="s")

    @functools.partial(
        pl.kernel,
        mesh=mesh,
        compiler_params=pltpu.CompilerParams(use_tc_tiling_on_sc=True,
                                             needs_layout_passes=False),
        out_type=jax.ShapeDtypeStruct((LATENT, BATCH), jnp.float32),
        scratch_types=[
            pltpu.VMEM((b_per_w,), jnp.int32),
            pltpu.VMEM((n_chunks, 128), jnp.int32),
            pltpu.VMEM((b_per_w, 128), jnp.float32),
            pltpu.VMEM((LATENT, b_per_w), jnp.float32),
            pltpu.SemaphoreType.DMA,
        ],
    )
    def gather(idx_hbm, pk_hbm, out_hbm, idx_v, r_v, rows_v, stage_v, sem):
        wid = lax.axis_index("s") * nc + lax.axis_index("c")
        base = pl.multiple_of(wid * b_per_w, b_per_w)
        lanes = lax.iota(jnp.int32, 16)

        pltpu.sync_copy(idx_hbm.at[pl.ds(base, b_per_w)], idx_v)
        for c in range(b_per_w // 16):
            rv = idx_v[pl.ds(c * 16, 16)] & (VP - 1)
            r_v[c // 8, pl.ds((c % 8) * 16, 16)] = rv
        copies = [
            pltpu.async_copy(pk_hbm.at[r_v.at[j]],
                             rows_v.at[pl.ds(j * 128, 128)], sem)
            for j in range(n_chunks)
        ]
        for cp in copies:
            cp.wait()

        def body(jg, carry):
            j_vec = jg * 16 + lanes
            s_vec = lax.shift_right_logical(idx_v[pl.ds(jg * 16, 16)], SBITS)
            c_base = s_vec * 16
            for f2 in range(16):
                vals = plsc.load_gather(rows_v, [j_vec, c_base + f2])
                bits = lax.bitcast_convert_type(vals, jnp.uint32)
                lo = lax.bitcast_convert_type(bits << 16, jnp.float32)
                hi = lax.bitcast_convert_type(
                    bits & jnp.uint32(0xFFFF0000), jnp.float32)
                stage_v[f2, pl.ds(jg * 16, 16)] = lo
                stage_v[f2 + 16, pl.ds(jg * 16, 16)] = hi
            return carry

        lax.fori_loop(0, b_per_w // 16, body, None)
        pltpu.sync_copy(stage_v, out_hbm.at[:, pl.ds(base, b_per_w)])

    return gather


_GATHER = _build_gather()


def _mlp_body(uet_ref, iet_ref, w1a_ref, w1b_ref, b1_ref, gamma_ref, beta_ref,
              w2_ref, b2_ref, out_ref):
    dnums = (((0,), (0,)), ((), ()))
    h = (lax.dot_general(uet_ref[...].astype(jnp.bfloat16),
                         w1a_ref[...].astype(jnp.bfloat16), dnums,
                         preferred_element_type=jnp.float32)
         + lax.dot_general(iet_ref[...].astype(jnp.bfloat16),
                           w1b_ref[...].astype(jnp.bfloat16), dnums,
                           preferred_element_type=jnp.float32)
         + b1_ref[...])
    mean = jnp.mean(h, axis=0, keepdims=True)
    var = jnp.mean((h - mean) ** 2, axis=0, keepdims=True)
    hn = (h - mean) * lax.rsqrt(var + 1e-5) * gamma_ref[...] + beta_ref[...]
    hr = jnp.maximum(hn, 0.0)
    logit = lax.dot_general(
        w2_ref[...], hr, (((0,), (1,)), ((), ())),
        preferred_element_type=jnp.float32) + b2_ref[...]
    out_ref[...] = jax.nn.sigmoid(logit)


_MLP = pl.pallas_call(
    _mlp_body,
    out_shape=jax.ShapeDtypeStruct((1, BATCH), jnp.float32),
)


def kernel(user, item, user_table, item_table, W1, b1, gamma, beta, W2, b2):
    ut, it = user_table.T, item_table.T
    upk = _REPACK(*([ut] * NS_GROUP))
    uet = _GATHER(user.astype(jnp.int32), upk)
    ipk = _REPACK(*([it] * NS_GROUP))
    iet = _GATHER(item.astype(jnp.int32), ipk)
    out = _MLP(
        uet, iet,
        W1[:LATENT], W1[LATENT:],
        b1.reshape(1, HIDDEN), gamma.reshape(1, HIDDEN), beta.reshape(1, HIDDEN),
        W2, b2.reshape(1, 1),
    )
    return out.reshape(BATCH)


# speedup vs baseline: 1.0791x; 1.0791x over previous
import functools

import jax
import jax.numpy as jnp
from jax import lax
from jax.experimental import pallas as pl
from jax.experimental.pallas import tpu as pltpu
from jax.experimental.pallas import tpu_sc as plsc

BATCH = 16384
LATENT = 32
HIDDEN = 128
NUM_ROWS = 1000000
VP = 131072
SBITS = 17
NS_GROUP = 8
RBLK = 4096
NBLK = VP // RBLK
MAXBLK = (NUM_ROWS - 1) // RBLK


def _repack_body(*refs):
    urefs, irefs = refs[:NS_GROUP], refs[NS_GROUP:2 * NS_GROUP]
    uout, iout = refs[2 * NS_GROUP], refs[2 * NS_GROUP + 1]
    k = NS_GROUP * LATENT
    frow = lax.broadcasted_iota(jnp.int32, (k, 128), 0)
    jcol = lax.broadcasted_iota(jnp.int32, (k, 128), 1)
    s = frow // LATENT
    f = frow % LATENT
    e_lo = ((f < 16) & (jcol == s * 16 + f)).astype(jnp.bfloat16)
    e_hi = ((f >= 16) & (jcol == s * 16 + f - 16)).astype(jnp.bfloat16)
    dnums = (((0,), (0,)), ((), ()))

    def pack(srefs):
        x = jnp.concatenate([r[...] for r in srefs],
                            axis=0).astype(jnp.bfloat16)
        lo = lax.dot_general(x, e_lo, dnums,
                             preferred_element_type=jnp.float32)
        hi = lax.dot_general(x, e_hi, dnums,
                             preferred_element_type=jnp.float32)
        lo_b = lax.bitcast_convert_type(lo, jnp.uint32)
        hi_b = lax.bitcast_convert_type(hi, jnp.uint32)
        pair = (hi_b & jnp.uint32(0xFFFF0000)) | (lo_b >> 16)
        return lax.bitcast_convert_type(pair, jnp.float32)

    uout[...] = pack(urefs)
    iout[...] = pack(irefs)


_REPACK = pl.pallas_call(
    _repack_body,
    grid=(NBLK,),
    in_specs=[pl.BlockSpec(
        (LATENT, RBLK),
        lambda i, s=s: (0, jnp.minimum(s * NBLK + i, MAXBLK)))
        for s in range(NS_GROUP)] * 2,
    out_specs=[pl.BlockSpec((RBLK, 128), lambda i: (i, 0))] * 2,
    out_shape=[jax.ShapeDtypeStruct((VP, 128), jnp.float32)] * 2,
)


def _build_gather():
    info = plsc.get_sparse_core_info()
    nc, ns = info.num_cores, info.num_subcores
    nw = nc * ns
    b_per_w = BATCH // nw
    n_chunks = b_per_w // 128
    mesh = plsc.VectorSubcoreMesh(core_axis_name="c", subcore_axis_name="s")

    @functools.partial(
        pl.kernel,
        mesh=mesh,
        compiler_params=pltpu.CompilerParams(use_tc_tiling_on_sc=True,
                                             needs_layout_passes=False),
        out_type=[
            jax.ShapeDtypeStruct((LATENT, BATCH), jnp.float32),
            jax.ShapeDtypeStruct((LATENT, BATCH), jnp.float32),
        ],
        scratch_types=[
            pltpu.VMEM((b_per_w,), jnp.int32),
            pltpu.VMEM((n_chunks, 128), jnp.int32),
            pltpu.VMEM((b_per_w, 128), jnp.float32),
            pltpu.VMEM((LATENT, b_per_w), jnp.float32),
            pltpu.SemaphoreType.DMA,
        ],
    )
    def gather(user_hbm, item_hbm, upk_hbm, ipk_hbm, ue_out, ie_out,
               idx_v, r_v, rows_v, stage_v, sem):
        wid = lax.axis_index("s") * nc + lax.axis_index("c")
        base = pl.multiple_of(wid * b_per_w, b_per_w)
        lanes = lax.iota(jnp.int32, 16)

        def one_table(idx_hbm, pk_hbm, out_hbm):
            pltpu.sync_copy(idx_hbm.at[pl.ds(base, b_per_w)], idx_v)
            for c in range(b_per_w // 16):
                rv = idx_v[pl.ds(c * 16, 16)] & (VP - 1)
                r_v[c // 8, pl.ds((c % 8) * 16, 16)] = rv
            copies = [
                pltpu.async_copy(pk_hbm.at[r_v.at[j]],
                                 rows_v.at[pl.ds(j * 128, 128)], sem)
                for j in range(n_chunks)
            ]
            for cp in copies:
                cp.wait()

            def body(jg, carry):
                j_vec = jg * 16 + lanes
                s_vec = lax.shift_right_logical(idx_v[pl.ds(jg * 16, 16)],
                                                SBITS)
                c_base = s_vec * 16
                for f2 in range(16):
                    vals = plsc.load_gather(rows_v, [j_vec, c_base + f2])
                    bits = lax.bitcast_convert_type(vals, jnp.uint32)
                    lo = lax.bitcast_convert_type(bits << 16, jnp.float32)
                    hi = lax.bitcast_convert_type(
                        bits & jnp.uint32(0xFFFF0000), jnp.float32)
                    stage_v[f2, pl.ds(jg * 16, 16)] = lo
                    stage_v[f2 + 16, pl.ds(jg * 16, 16)] = hi
                return carry

            lax.fori_loop(0, b_per_w // 16, body, None)
            pltpu.sync_copy(stage_v, out_hbm.at[:, pl.ds(base, b_per_w)])

        one_table(user_hbm, upk_hbm, ue_out)
        one_table(item_hbm, ipk_hbm, ie_out)

    return gather


_GATHER = _build_gather()


def _mlp_body(uet_ref, iet_ref, w1a_ref, w1b_ref, b1_ref, gamma_ref, beta_ref,
              w2_ref, b2_ref, out_ref):
    dnums = (((0,), (0,)), ((), ()))
    h = (lax.dot_general(uet_ref[...].astype(jnp.bfloat16),
                         w1a_ref[...].astype(jnp.bfloat16), dnums,
                         preferred_element_type=jnp.float32)
         + lax.dot_general(iet_ref[...].astype(jnp.bfloat16),
                           w1b_ref[...].astype(jnp.bfloat16), dnums,
                           preferred_element_type=jnp.float32)
         + b1_ref[...])
    mean = jnp.mean(h, axis=0, keepdims=True)
    var = jnp.mean((h - mean) ** 2, axis=0, keepdims=True)
    hn = (h - mean) * lax.rsqrt(var + 1e-5) * gamma_ref[...] + beta_ref[...]
    hr = jnp.maximum(hn, 0.0)
    logit = lax.dot_general(
        w2_ref[...], hr, (((0,), (1,)), ((), ())),
        preferred_element_type=jnp.float32) + b2_ref[...]
    out_ref[...] = jax.nn.sigmoid(logit)


_MLP = pl.pallas_call(
    _mlp_body,
    out_shape=jax.ShapeDtypeStruct((1, BATCH), jnp.float32),
)


def kernel(user, item, user_table, item_table, W1, b1, gamma, beta, W2, b2):
    ut, it = user_table.T, item_table.T
    upk, ipk = _REPACK(*([ut] * NS_GROUP), *([it] * NS_GROUP))
    uet, iet = _GATHER(user.astype(jnp.int32), item.astype(jnp.int32),
                       upk, ipk)
    out = _MLP(
        uet, iet,
        W1[:LATENT], W1[LATENT:],
        b1.reshape(1, HIDDEN), gamma.reshape(1, HIDDEN), beta.reshape(1, HIDDEN),
        W2, b2.reshape(1, 1),
    )
    return out.reshape(BATCH)
